# baseline (speedup 1.0000x reference)
import functools

import jax
import jax.numpy as jnp
import numpy as np
from jax import lax
from jax.experimental import pallas as pl
from jax.experimental.pallas import tpu as pltpu
from jax.experimental.pallas import tpu_sc as plsc

_DISTS = (0.25, 0.5, 1.0, 2.0)
_NSAMPLES = (32, 64, 64, 128)

_ROWS = 2048


def _mm_stats_kernel(x_ref, st_ref, w_ref, y_ref, s_ref, *, nsteps, norm):
    i = pl.program_id(0)
    x = x_ref[...]
    if norm:
        mean = st_ref[0:1, :]
        rstd = st_ref[1:2, :]
        x = jnp.maximum((x - mean) * rstd, 0.0)
    y = jnp.dot(x, w_ref[...], preferred_element_type=jnp.float32)
    y_ref[...] = y
    blk = jnp.concatenate(
        [jnp.sum(y, axis=0, keepdims=True),
         jnp.sum(y * y, axis=0, keepdims=True)], axis=0)

    @pl.when(i == 0)
    def _():
        s_ref[...] = blk

    @pl.when(i > 0)
    def _():
        s_ref[...] = s_ref[...] + blk


def _mm_stats(x, stats, wt):
    P, cin = x.shape
    cout = wt.shape[1]
    nsteps = P // _ROWS
    norm = stats is not None
    if stats is None:
        stats = jnp.zeros((2, cin), jnp.float32)
    kern = functools.partial(_mm_stats_kernel, nsteps=nsteps, norm=norm)
    y, s = pl.pallas_call(
        kern,
        grid=(nsteps,),
        in_specs=[
            pl.BlockSpec((_ROWS, cin), lambda i: (i, 0)),
            pl.BlockSpec((2, cin), lambda i: (0, 0)),
            pl.BlockSpec((cin, cout), lambda i: (0, 0)),
        ],
        out_specs=[
            pl.BlockSpec((_ROWS, cout), lambda i: (i, 0)),
            pl.BlockSpec((2, cout), lambda i: (0, 0)),
        ],
        out_shape=[
            jax.ShapeDtypeStruct((P, cout), jnp.float32),
            jax.ShapeDtypeStruct((2, cout), jnp.float32),
        ],
    )(x, stats, wt)
    return y, s


def _final_kernel(y_ref, st_ref, v_ref, o_ref, *, k):
    mean = st_ref[0:1, :]
    rstd = st_ref[1:2, :]
    x = jnp.maximum((y_ref[...] - mean) * rstd, 0.0)
    r, c = x.shape
    x = jnp.max(x.reshape(r // k, k, c), axis=1)
    o_ref[...] = x * v_ref[...]


def _mlp_final(y, stats, valid, k):
    P, c = y.shape
    nsteps = P // _ROWS
    rk = _ROWS // k
    kern = functools.partial(_final_kernel, k=k)
    return pl.pallas_call(
        kern,
        grid=(nsteps,),
        in_specs=[
            pl.BlockSpec((_ROWS, c), lambda i: (i, 0)),
            pl.BlockSpec((2, c), lambda i: (0, 0)),
            pl.BlockSpec((rk, 1), lambda i: (i, 0)),
        ],
        out_specs=pl.BlockSpec((rk, c), lambda i: (i, 0)),
        out_shape=jax.ShapeDtypeStruct((P // k, c), jnp.float32),
    )(y, stats, valid)


def _to_meanrstd(s, n):
    mean = s[0] / n
    var = jnp.maximum(s[1] / n - mean * mean, 0.0)
    return jnp.stack([mean, jax.lax.rsqrt(var + 1e-5)])


def _mlp_chain(g, valid, w1t, w2t, w3t, k):
    P = g.shape[0]
    y1, s1 = _mm_stats(g, None, w1t)
    y2, s2 = _mm_stats(y1, _to_meanrstd(s1, P), w2t)
    y3, s3 = _mm_stats(y2, _to_meanrstd(s2, P), w3t)
    return _mlp_final(y3, _to_meanrstd(s3, P), valid, k)


def _sc_ball_gather(px, py, pz, qx, qy, qz, B, N, M, k, dist):
    P = B * M * k
    mhalf = M // 2
    ngroups = mhalf // 16
    mesh = plsc.VectorSubcoreMesh(core_axis_name="c", subcore_axis_name="s")

    @functools.partial(
        pl.kernel,
        mesh=mesh,
        compiler_params=pltpu.CompilerParams(
            use_tc_tiling_on_sc=False, needs_layout_passes=False),
        out_type=[
            jax.ShapeDtypeStruct((P * 3,), jnp.float32),
            jax.ShapeDtypeStruct((B * M,), jnp.float32),
        ],
        scratch_types=[
            pltpu.VMEM((N,), jnp.float32),
            pltpu.VMEM((N,), jnp.float32),
            pltpu.VMEM((N,), jnp.float32),
            pltpu.VMEM((mhalf,), jnp.float32),
            pltpu.VMEM((mhalf,), jnp.float32),
            pltpu.VMEM((mhalf,), jnp.float32),
            pltpu.VMEM((16 * k,), jnp.int32),
            pltpu.VMEM((16 * k * 3,), jnp.float32),
            pltpu.VMEM((16,), jnp.float32),
        ],
    )
    def sck(px_h, py_h, pz_h, qx_h, qy_h, qz_h, g_h, valid_h,
            xb, yb, zb, qxb, qyb, qzb, idxb, gout, validb):
        w = lax.axis_index("s") * 2 + lax.axis_index("c")
        b = w // 2
        m0base = b * M + (w % 2) * mhalf
        pltpu.sync_copy(px_h.at[pl.ds(b * N, N)], xb)
        pltpu.sync_copy(py_h.at[pl.ds(b * N, N)], yb)
        pltpu.sync_copy(pz_h.at[pl.ds(b * N, N)], zb)
        pltpu.sync_copy(qx_h.at[pl.ds(m0base, mhalf)], qxb)
        pltpu.sync_copy(qy_h.at[pl.ds(m0base, mhalf)], qyb)
        pltpu.sync_copy(qz_h.at[pl.ds(m0base, mhalf)], qzb)
        lanes = lax.iota(jnp.int32, 16)
        lk = lanes * k

        def group_body(gi, _):
            zq = qzb[pl.ds(gi * 16, 16)]
            qxv = qxb[pl.ds(gi * 16, 16)]
            qyv = qyb[pl.ds(gi * 16, 16)]

            def scan_body(n, cnt):
                nv = jnp.full((16,), n, jnp.int32)
                zn = plsc.load_gather(zb, [nv])
                hit = jnp.abs(zn - zq) < dist
                wm = jnp.logical_and(hit, cnt < k)
                plsc.store_scatter(idxb, [lk + cnt], nv, mask=wm)
                return cnt + jnp.where(wm, 1, 0)

            cnt = lax.fori_loop(0, N, scan_body,
                                jnp.zeros((16,), jnp.int32), unroll=4)
            validb[...] = jnp.where(cnt > 0, 1.0, 0.0).astype(jnp.float32)
            pltpu.sync_copy(validb, valid_h.at[pl.ds(m0base + gi * 16, 16)])
            first = plsc.load_gather(idxb, [lk])
            first = jnp.where(cnt > 0, first, 0)

            def slot_body(si, _):
                sv = jnp.full((16,), si, jnp.int32)
                slot = plsc.load_gather(idxb, [lk + sv])
                sel = jnp.where(sv < cnt, slot, first)
                pxv = plsc.load_gather(xb, [sel])
                pyv = plsc.load_gather(yb, [sel])
                pzv = plsc.load_gather(zb, [sel])
                base3 = (lk + sv) * 3
                plsc.store_scatter(gout, [base3], pxv - qxv)
                plsc.store_scatter(gout, [base3 + 1], pyv - qyv)
                plsc.store_scatter(gout, [base3 + 2], pzv - zq)
                return 0

            lax.fori_loop(0, k, slot_body, 0)
            pltpu.sync_copy(
                gout, g_h.at[pl.ds((m0base + gi * 16) * k * 3, 16 * k * 3)])
            return 0

        lax.fori_loop(0, ngroups, group_body, 0)

    return sck(px, py, pz, qx, qy, qz)


def _pointnet_feat(pc, new_pc, w1, w2, w3, dist, k):
    B, _, M = new_pc.shape
    N = pc.shape[2]
    px, py, pz = (pc[:, c, :].reshape(-1) for c in range(3))
    qx, qy, qz = (new_pc[:, c, :].reshape(-1) for c in range(3))
    g, valid = _sc_ball_gather(px, py, pz, qx, qy, qz, B, N, M, k, dist)
    g = g.reshape(B * M * k, 3)
    valid = valid.reshape(B * M, 1)
    return _mlp_chain(g, valid, w1.T, w2.T, w3.T, k)


def _layer_kernel(*refs, n_in, terms, norm, has_bias):
    nt = len(terms)
    xr = refs[:n_in]
    sr = refs[n_in:2 * n_in]
    wr = refs[2 * n_in:2 * n_in + nt]
    pos = 2 * n_in + nt
    br = refs[pos] if has_bias else None
    y_ref, s_ref = refs[pos + int(has_bias)], refs[pos + int(has_bias) + 1]
    i = pl.program_id(0)
    a = []
    for j in range(n_in):
        x = xr[j][...]
        if norm[j]:
            x = jnp.maximum((x - sr[j][0:1, :]) * sr[j][1:2, :], 0.0)
        a.append(x)
    y = None
    for (ii, roll, mk), w_ref in zip(terms, wr):
        x = a[ii]
        if roll:
            x = pltpu.roll(x, roll % x.shape[0], 0)
        if mk is not None:
            rowid = lax.broadcasted_iota(jnp.int32, x.shape, 0)
            edge = 0 if mk == 'first' else x.shape[0] - 1
            x = jnp.where(rowid == edge, 0.0, x)
        t = jnp.dot(x, w_ref[...], preferred_element_type=jnp.float32)
        y = t if y is None else y + t
    if has_bias:
        y = y + br[...]
    y_ref[...] = y
    blk = jnp.concatenate(
        [jnp.sum(y, axis=0, keepdims=True),
         jnp.sum(y * y, axis=0, keepdims=True)], axis=0)

    @pl.when(i == 0)
    def _():
        s_ref[...] = blk

    @pl.when(i > 0)
    def _():
        s_ref[...] = s_ref[...] + blk


def _layer(inputs, stats, terms, wts, L, cout, bias=None):
    n_in = len(inputs)
    nb = inputs[0].shape[0] // L
    norm = tuple(s is not None for s in stats)
    stats = [s if s is not None else jnp.zeros((2, x.shape[1]), jnp.float32)
             for s, x in zip(stats, inputs)]
    has_bias = bias is not None
    in_specs = (
        [pl.BlockSpec((L, x.shape[1]), lambda i: (i, 0)) for x in inputs]
        + [pl.BlockSpec((2, s.shape[1]), lambda i: (0, 0)) for s in stats]
        + [pl.BlockSpec(w.shape, lambda i: (0, 0)) for w in wts]
        + ([pl.BlockSpec((1, cout), lambda i: (0, 0))] if has_bias else [])
    )
    kern = functools.partial(_layer_kernel, n_in=n_in, terms=tuple(terms),
                             norm=norm, has_bias=has_bias)
    args = list(inputs) + stats + list(wts) + ([bias] if has_bias else [])
    y, s = pl.pallas_call(
        kern,
        grid=(nb,),
        in_specs=in_specs,
        out_specs=[
            pl.BlockSpec((L, cout), lambda i: (i, 0)),
            pl.BlockSpec((2, cout), lambda i: (0, 0)),
        ],
        out_shape=[
            jax.ShapeDtypeStruct((nb * L, cout), jnp.float32),
            jax.ShapeDtypeStruct((2, cout), jnp.float32),
        ],
    )(*args)
    return y, s


_K3 = ((0, 1, 'first'), (0, 0, None), (0, -1, 'last'))


def _split2(y, B, L):
    C = y.shape[1]
    y3 = y.reshape(B, L // 2, 2, C)
    xe = y3[:, :, 0, :].reshape(B * L // 2, C)
    xo = y3[:, :, 1, :]
    xos = jnp.concatenate([jnp.zeros((B, 1, C), y.dtype), xo[:, :-1]], 1)
    return xe, xos.reshape(B * L // 2, C), xo.reshape(B * L // 2, C)


def _w3(p, name):
    return [p[name][:, :, t].T for t in range(3)]


def _trunk_heads(f1, f2, f3, f4, p):
    B = 16
    y1, s1 = _layer([f1], [None], _K3, _w3(p, 'b1c1'), 1024, 128)
    st1 = _to_meanrstd(s1, B * 1024)
    xe, xos, xo = _split2(y1, B, 1024)
    w0, w1t, w2t = _w3(p, 'b2c1')
    y2, s2 = _layer([xos, xe, xo], [st1, st1, st1],
                    [(0, 0, 'first'), (1, 0, None), (2, 0, None)],
                    [w0, w1t, w2t], 512, 128)
    st2 = _to_meanrstd(s2, B * 512)
    y3, s3 = _layer([y2], [st2], _K3, _w3(p, 'b2c2'), 512, 128)
    st3 = _to_meanrstd(s3, B * 512)
    y4, s4 = _layer([y3, f2], [st3, None], [(0, 0, None), (1, 0, None)],
                    [p['b2m'][:, :128, 0].T, p['b2m'][:, 128:, 0].T], 512, 128)
    st4 = _to_meanrstd(s4, B * 512)

    xe, xos, xo = _split2(y4, B, 512)
    w0, w1t, w2t = _w3(p, 'b3c1')
    y5, s5 = _layer([xos, xe, xo], [st4, st4, st4],
                    [(0, 0, 'first'), (1, 0, None), (2, 0, None)],
                    [w0, w1t, w2t], 256, 256)
    st5 = _to_meanrstd(s5, B * 256)
    y6, s6 = _layer([y5], [st5], _K3, _w3(p, 'b3c2'), 256, 256)
    st6 = _to_meanrstd(s6, B * 256)
    y7, s7 = _layer([y6, f3], [st6, None], [(0, 0, None), (1, 0, None)],
                    [p['b3m'][:, :256, 0].T, p['b3m'][:, 256:, 0].T], 256, 256)
    st7 = _to_meanrstd(s7, B * 256)

    xe, xos, xo = _split2(y7, B, 256)
    w0, w1t, w2t = _w3(p, 'b4c1')
    y8, s8 = _layer([xos, xe, xo], [st7, st7, st7],
                    [(0, 0, 'first'), (1, 0, None), (2, 0, None)],
                    [w0, w1t, w2t], 128, 512)
    st8 = _to_meanrstd(s8, B * 128)
    y9, s9 = _layer([y8], [st8], _K3, _w3(p, 'b4c2'), 128, 512)
    st9 = _to_meanrstd(s9, B * 128)
    y10, s10 = _layer([y9, f4], [st9, None], [(0, 0, None), (1, 0, None)],
                      [p['b4m'][:, :512, 0].T, p['b4m'][:, 512:, 0].T],
                      128, 512)
    st10 = _to_meanrstd(s10, B * 128)

    yd2, sd2 = _layer([y4], [st4], [(0, 0, None)], [p['d2'][:, :, 0]],
                      512, 256)
    std2 = _to_meanrstd(sd2, B * 512)
    d3cat = jnp.concatenate([p['d3'][:, :, 0], p['d3'][:, :, 1]], axis=1)
    yd3, sd3 = _layer([y7], [st7], [(0, 0, None)], [d3cat], 256, 512)
    std3 = _to_meanrstd(sd3[:, :256] + sd3[:, 256:], B * 512)
    yd3 = yd3.reshape(B * 512, 256)
    d4cat = jnp.concatenate([p['d4'][:, :, j] for j in range(4)], axis=1)
    yd4, sd4 = _layer([y10], [st10], [(0, 0, None)], [d4cat], 128, 1024)
    std4 = _to_meanrstd(sd4[:, :256] + sd4[:, 256:512]
                        + sd4[:, 512:768] + sd4[:, 768:], B * 512)
    yd4 = yd4.reshape(B * 512, 256)

    wh = jnp.concatenate([p['cls_w'][:, :, 0], p['reg_w'][:, :, 0]], 0).T
    bh = jnp.concatenate([p['cls_b'], p['reg_b']])[None, :]
    out, _ = _layer([yd2, yd3, yd4], [std2, std3, std4],
                    [(0, 0, None), (1, 0, None), (2, 0, None)],
                    [wh[:256], wh[256:512], wh[512:]], 512, 41, bias=bh)
    return out.reshape(B, 512, 41).transpose(0, 2, 1)


def kernel(point_cloud, pc1, pc2, pc3, pc4, one_hot_vec, params):
    p = params
    pcs = (pc1, pc2, pc3, pc4)
    B = point_cloud.shape[0]
    feats = []
    for i, (dist, k) in enumerate(zip(_DISTS, _NSAMPLES), start=1):
        f = _pointnet_feat(point_cloud, pcs[i - 1], p['pn%d_w1' % i],
                           p['pn%d_w2' % i], p['pn%d_w3' % i], dist, k)
        M = pcs[i - 1].shape[2]
        c3 = f.shape[1]
        oh = jnp.broadcast_to(one_hot_vec[:, None, :], (B, M, 3))
        feats.append(jnp.concatenate([f.reshape(B, M, c3), oh],
                                     axis=2).reshape(B * M, c3 + 3))
    return _trunk_heads(*feats, p)


# speedup vs baseline: 1.0660x; 1.0660x over previous
import functools

import jax
import jax.numpy as jnp
import numpy as np
from jax import lax
from jax.experimental import pallas as pl
from jax.experimental.pallas import tpu as pltpu
from jax.experimental.pallas import tpu_sc as plsc

_DISTS = (0.25, 0.5, 1.0, 2.0)
_NSAMPLES = (32, 64, 64, 128)

_ROWS = 2048


def _mm_stats_kernel(x_ref, st_ref, w_ref, y_ref, s_ref, *, nsteps, norm):
    i = pl.program_id(0)
    x = x_ref[...]
    if norm:
        mean = st_ref[0:1, :]
        rstd = st_ref[1:2, :]
        x = jnp.maximum((x - mean) * rstd, 0.0)
    y = jnp.dot(x, w_ref[...], preferred_element_type=jnp.float32)
    y_ref[...] = y
    blk = jnp.concatenate(
        [jnp.sum(y, axis=0, keepdims=True),
         jnp.sum(y * y, axis=0, keepdims=True)], axis=0)

    @pl.when(i == 0)
    def _():
        s_ref[...] = blk

    @pl.when(i > 0)
    def _():
        s_ref[...] = s_ref[...] + blk


def _mm_stats(x, stats, wt):
    P, cin = x.shape
    cout = wt.shape[1]
    nsteps = P // _ROWS
    norm = stats is not None
    if stats is None:
        stats = jnp.zeros((2, cin), jnp.float32)
    kern = functools.partial(_mm_stats_kernel, nsteps=nsteps, norm=norm)
    y, s = pl.pallas_call(
        kern,
        grid=(nsteps,),
        in_specs=[
            pl.BlockSpec((_ROWS, cin), lambda i: (i, 0)),
            pl.BlockSpec((2, cin), lambda i: (0, 0)),
            pl.BlockSpec((cin, cout), lambda i: (0, 0)),
        ],
        out_specs=[
            pl.BlockSpec((_ROWS, cout), lambda i: (i, 0)),
            pl.BlockSpec((2, cout), lambda i: (0, 0)),
        ],
        out_shape=[
            jax.ShapeDtypeStruct((P, cout), jnp.float32),
            jax.ShapeDtypeStruct((2, cout), jnp.float32),
        ],
    )(x, stats, wt)
    return y, s


def _final_kernel(y_ref, st_ref, v_ref, o_ref, *, k):
    mean = st_ref[0:1, :]
    rstd = st_ref[1:2, :]
    x = jnp.maximum((y_ref[...] - mean) * rstd, 0.0)
    r, c = x.shape
    x = jnp.max(x.reshape(r // k, k, c), axis=1)
    o_ref[...] = x * v_ref[...]


def _mlp_final(y, stats, valid, k):
    P, c = y.shape
    nsteps = P // _ROWS
    rk = _ROWS // k
    kern = functools.partial(_final_kernel, k=k)
    return pl.pallas_call(
        kern,
        grid=(nsteps,),
        in_specs=[
            pl.BlockSpec((_ROWS, c), lambda i: (i, 0)),
            pl.BlockSpec((2, c), lambda i: (0, 0)),
            pl.BlockSpec((rk, 1), lambda i: (i, 0)),
        ],
        out_specs=pl.BlockSpec((rk, c), lambda i: (i, 0)),
        out_shape=jax.ShapeDtypeStruct((P // k, c), jnp.float32),
    )(y, stats, valid)


def _to_meanrstd(s, n):
    mean = s[0] / n
    var = jnp.maximum(s[1] / n - mean * mean, 0.0)
    return jnp.stack([mean, jax.lax.rsqrt(var + 1e-5)])


def _mlp_chain(g, valid, w1t, w2t, w3t, k):
    P = g.shape[0]
    y1, s1 = _mm_stats(g, None, w1t)
    y2, s2 = _mm_stats(y1, _to_meanrstd(s1, P), w2t)
    y3, s3 = _mm_stats(y2, _to_meanrstd(s2, P), w3t)
    return _mlp_final(y3, _to_meanrstd(s3, P), valid, k)


def _sc_ball_gather(px, py, pz, qx, qy, qz, B, N, M, k, dist):
    P = B * M * k
    mhalf = M // 2
    ngroups = mhalf // 16
    mesh = plsc.VectorSubcoreMesh(core_axis_name="c", subcore_axis_name="s")

    @functools.partial(
        pl.kernel,
        mesh=mesh,
        compiler_params=pltpu.CompilerParams(
            use_tc_tiling_on_sc=False, needs_layout_passes=False),
        out_type=[
            jax.ShapeDtypeStruct((P * 3,), jnp.float32),
            jax.ShapeDtypeStruct((B * M,), jnp.float32),
        ],
        scratch_types=[
            pltpu.VMEM((N,), jnp.float32),
            pltpu.VMEM((N,), jnp.float32),
            pltpu.VMEM((N,), jnp.float32),
            pltpu.VMEM((mhalf,), jnp.float32),
            pltpu.VMEM((mhalf,), jnp.float32),
            pltpu.VMEM((mhalf,), jnp.float32),
            pltpu.VMEM((16 * k,), jnp.int32),
            pltpu.VMEM((16 * k * 3,), jnp.float32),
            pltpu.VMEM((16,), jnp.float32),
        ],
    )
    def sck(px_h, py_h, pz_h, qx_h, qy_h, qz_h, g_h, valid_h,
            xb, yb, zb, qxb, qyb, qzb, idxb, gout, validb):
        w = lax.axis_index("s") * 2 + lax.axis_index("c")
        b = w // 2
        m0base = b * M + (w % 2) * mhalf
        pltpu.sync_copy(px_h.at[pl.ds(b * N, N)], xb)
        pltpu.sync_copy(py_h.at[pl.ds(b * N, N)], yb)
        pltpu.sync_copy(pz_h.at[pl.ds(b * N, N)], zb)
        pltpu.sync_copy(qx_h.at[pl.ds(m0base, mhalf)], qxb)
        pltpu.sync_copy(qy_h.at[pl.ds(m0base, mhalf)], qyb)
        pltpu.sync_copy(qz_h.at[pl.ds(m0base, mhalf)], qzb)
        lanes = lax.iota(jnp.int32, 16)
        lk = lanes * k

        def group_body(gi, _):
            zq = qzb[pl.ds(gi * 16, 16)]
            qxv = qxb[pl.ds(gi * 16, 16)]
            qyv = qyb[pl.ds(gi * 16, 16)]

            def scan_body(n, cnt):
                nv = jnp.full((16,), n, jnp.int32)
                zn = plsc.load_gather(zb, [nv])
                hit = jnp.abs(zn - zq) < dist
                wm = jnp.logical_and(hit, cnt < k)
                plsc.store_scatter(idxb, [lk + cnt], nv, mask=wm)
                return cnt + jnp.where(wm, 1, 0)

            cnt = lax.fori_loop(0, N, scan_body,
                                jnp.zeros((16,), jnp.int32), unroll=4)
            validb[...] = jnp.where(cnt > 0, 1.0, 0.0).astype(jnp.float32)
            pltpu.sync_copy(validb, valid_h.at[pl.ds(m0base + gi * 16, 16)])
            first = plsc.load_gather(idxb, [lk])
            first = jnp.where(cnt > 0, first, 0)

            def slot_body(si, _):
                sv = jnp.full((16,), si, jnp.int32)
                slot = plsc.load_gather(idxb, [lk + sv])
                sel = jnp.where(sv < cnt, slot, first)
                pxv = plsc.load_gather(xb, [sel])
                pyv = plsc.load_gather(yb, [sel])
                pzv = plsc.load_gather(zb, [sel])
                base3 = (lk + sv) * 3
                plsc.store_scatter(gout, [base3], pxv - qxv)
                plsc.store_scatter(gout, [base3 + 1], pyv - qyv)
                plsc.store_scatter(gout, [base3 + 2], pzv - zq)
                return 0

            lax.fori_loop(0, k, slot_body, 0)
            pltpu.sync_copy(
                gout, g_h.at[pl.ds((m0base + gi * 16) * k * 3, 16 * k * 3)])
            return 0

        lax.fori_loop(0, ngroups, group_body, 0)

    return sck(px, py, pz, qx, qy, qz)


def _pointnet_feat(pc, new_pc, w1, w2, w3, dist, k):
    B, _, M = new_pc.shape
    N = pc.shape[2]
    px, py, pz = (pc[:, c, :].reshape(-1) for c in range(3))
    qx, qy, qz = (new_pc[:, c, :].reshape(-1) for c in range(3))
    g, valid = _sc_ball_gather(px, py, pz, qx, qy, qz, B, N, M, k, dist)
    g = g.reshape(B * M * k, 3)
    valid = valid.reshape(B * M, 1)
    return _mlp_chain(g, valid, w1.T, w2.T, w3.T, k)


def _layer_kernel(*refs, n_in, terms, norm, has_bias, L, counts):
    nt = len(terms)
    xr = refs[:n_in]
    sr = refs[n_in:2 * n_in]
    wr = refs[2 * n_in:2 * n_in + nt]
    pos = 2 * n_in + nt
    br = refs[pos] if has_bias else None
    y_ref, s_ref = refs[pos + int(has_bias)], refs[pos + int(has_bias) + 1]
    a = []
    for j in range(n_in):
        x = xr[j][...]
        if norm[j]:
            s = sr[j][...]
            mean = s[0:1, :] * (1.0 / counts[j])
            var = jnp.maximum(s[1:2, :] * (1.0 / counts[j]) - mean * mean, 0.0)
            x = jnp.maximum((x - mean) * lax.rsqrt(var + 1e-5), 0.0)
        a.append(x)
    y = None
    for (ii, roll, mk), w_ref in zip(terms, wr):
        x = a[ii]
        if roll:
            x = pltpu.roll(x, roll % x.shape[0], 0)
        if mk is not None:
            rowid = lax.broadcasted_iota(jnp.int32, x.shape, 0)
            edge = 0 if mk == 'first' else L - 1
            x = jnp.where(rowid % L == edge, 0.0, x)
        t = jnp.dot(x, w_ref[...], preferred_element_type=jnp.float32)
        y = t if y is None else y + t
    if has_bias:
        y = y + br[...]
    y_ref[...] = y
    s_ref[...] = jnp.concatenate(
        [jnp.sum(y, axis=0, keepdims=True),
         jnp.sum(y * y, axis=0, keepdims=True)], axis=0)


def _layer(inputs, stats, terms, wts, L, cout, bias=None):
    n_in = len(inputs)
    rows = inputs[0].shape[0]
    norm = tuple(s is not None for s in stats)
    counts = tuple(float(s[1]) if s is not None else 1.0 for s in stats)
    stats = [s[0] if s is not None else jnp.zeros((2, x.shape[1]), jnp.float32)
             for s, x in zip(stats, inputs)]
    has_bias = bias is not None
    in_specs = (
        [pl.BlockSpec(x.shape, lambda i: (0, 0)) for x in inputs]
        + [pl.BlockSpec(s.shape, lambda i: (0, 0)) for s in stats]
        + [pl.BlockSpec(w.shape, lambda i: (0, 0)) for w in wts]
        + ([pl.BlockSpec((1, cout), lambda i: (0, 0))] if has_bias else [])
    )
    kern = functools.partial(_layer_kernel, n_in=n_in, terms=tuple(terms),
                             norm=norm, has_bias=has_bias, L=L, counts=counts)
    args = list(inputs) + stats + list(wts) + ([bias] if has_bias else [])
    y, s = pl.pallas_call(
        kern,
        grid=(1,),
        in_specs=in_specs,
        out_specs=[
            pl.BlockSpec((rows, cout), lambda i: (0, 0)),
            pl.BlockSpec((2, cout), lambda i: (0, 0)),
        ],
        out_shape=[
            jax.ShapeDtypeStruct((rows, cout), jnp.float32),
            jax.ShapeDtypeStruct((2, cout), jnp.float32),
        ],
    )(*args)
    return y, s


_K3 = ((0, 1, 'first'), (0, 0, None), (0, -1, 'last'))


def _split2(y, B, L):
    C = y.shape[1]
    y3 = y.reshape(B, L // 2, 2, C)
    xe = y3[:, :, 0, :].reshape(B * L // 2, C)
    xo = y3[:, :, 1, :]
    xos = jnp.concatenate([jnp.zeros((B, 1, C), y.dtype), xo[:, :-1]], 1)
    return xe, xos.reshape(B * L // 2, C), xo.reshape(B * L // 2, C)


def _w3(p, name):
    return [p[name][:, :, t].T for t in range(3)]


def _trunk_heads(f1, f2, f3, f4, p):
    B = 16
    y1, s1 = _layer([f1], [None], _K3, _w3(p, 'b1c1'), 1024, 128)
    st1 = (s1, B * 1024)
    xe, xos, xo = _split2(y1, B, 1024)
    w0, w1t, w2t = _w3(p, 'b2c1')
    y2, s2 = _layer([xos, xe, xo], [st1, st1, st1],
                    [(0, 0, 'first'), (1, 0, None), (2, 0, None)],
                    [w0, w1t, w2t], 512, 128)
    st2 = (s2, B * 512)
    y3, s3 = _layer([y2], [st2], _K3, _w3(p, 'b2c2'), 512, 128)
    st3 = (s3, B * 512)
    y4, s4 = _layer([y3, f2], [st3, None], [(0, 0, None), (1, 0, None)],
                    [p['b2m'][:, :128, 0].T, p['b2m'][:, 128:, 0].T], 512, 128)
    st4 = (s4, B * 512)

    xe, xos, xo = _split2(y4, B, 512)
    w0, w1t, w2t = _w3(p, 'b3c1')
    y5, s5 = _layer([xos, xe, xo], [st4, st4, st4],
                    [(0, 0, 'first'), (1, 0, None), (2, 0, None)],
                    [w0, w1t, w2t], 256, 256)
    st5 = (s5, B * 256)
    y6, s6 = _layer([y5], [st5], _K3, _w3(p, 'b3c2'), 256, 256)
    st6 = (s6, B * 256)
    y7, s7 = _layer([y6, f3], [st6, None], [(0, 0, None), (1, 0, None)],
                    [p['b3m'][:, :256, 0].T, p['b3m'][:, 256:, 0].T], 256, 256)
    st7 = (s7, B * 256)

    xe, xos, xo = _split2(y7, B, 256)
    w0, w1t, w2t = _w3(p, 'b4c1')
    y8, s8 = _layer([xos, xe, xo], [st7, st7, st7],
                    [(0, 0, 'first'), (1, 0, None), (2, 0, None)],
                    [w0, w1t, w2t], 128, 512)
    st8 = (s8, B * 128)
    y9, s9 = _layer([y8], [st8], _K3, _w3(p, 'b4c2'), 128, 512)
    st9 = (s9, B * 128)
    y10, s10 = _layer([y9, f4], [st9, None], [(0, 0, None), (1, 0, None)],
                      [p['b4m'][:, :512, 0].T, p['b4m'][:, 512:, 0].T],
                      128, 512)
    st10 = (s10, B * 128)

    yd2, sd2 = _layer([y4], [st4], [(0, 0, None)], [p['d2'][:, :, 0]],
                      512, 256)
    std2 = (sd2, B * 512)
    d3cat = jnp.concatenate([p['d3'][:, :, 0], p['d3'][:, :, 1]], axis=1)
    yd3, sd3 = _layer([y7], [st7], [(0, 0, None)], [d3cat], 256, 512)
    std3 = (sd3[:, :256] + sd3[:, 256:], B * 512)
    yd3 = yd3.reshape(B * 512, 256)
    d4cat = jnp.concatenate([p['d4'][:, :, j] for j in range(4)], axis=1)
    yd4, sd4 = _layer([y10], [st10], [(0, 0, None)], [d4cat], 128, 1024)
    std4 = (sd4[:, :256] + sd4[:, 256:512]
            + sd4[:, 512:768] + sd4[:, 768:], B * 512)
    yd4 = yd4.reshape(B * 512, 256)

    wh = jnp.concatenate([p['cls_w'][:, :, 0], p['reg_w'][:, :, 0]], 0).T
    bh = jnp.concatenate([p['cls_b'], p['reg_b']])[None, :]
    out, _ = _layer([yd2, yd3, yd4], [std2, std3, std4],
                    [(0, 0, None), (1, 0, None), (2, 0, None)],
                    [wh[:256], wh[256:512], wh[512:]], 512, 41, bias=bh)
    return out.reshape(B, 512, 41).transpose(0, 2, 1)


def kernel(point_cloud, pc1, pc2, pc3, pc4, one_hot_vec, params):
    p = params
    pcs = (pc1, pc2, pc3, pc4)
    B = point_cloud.shape[0]
    feats = []
    for i, (dist, k) in enumerate(zip(_DISTS, _NSAMPLES), start=1):
        f = _pointnet_feat(point_cloud, pcs[i - 1], p['pn%d_w1' % i],
                           p['pn%d_w2' % i], p['pn%d_w3' % i], dist, k)
        M = pcs[i - 1].shape[2]
        c3 = f.shape[1]
        oh = jnp.broadcast_to(one_hot_vec[:, None, :], (B, M, 3))
        feats.append(jnp.concatenate([f.reshape(B, M, c3), oh],
                                     axis=2).reshape(B * M, c3 + 3))
    return _trunk_heads(*feats, p)
